# baseline (speedup 1.0000x reference)
import jax
import jax.numpy as jnp
import numpy as np
from jax import lax
from jax.experimental import pallas as pl
from jax.experimental.pallas import tpu as pltpu
from jax.experimental.pallas import tpu_sc as plsc

MAXLEN = 200
EMBED_DIM = 64
SCALE = 8.0

NC = 2
NS = 16
NW = NC * NS

B = 4096
BCH = B // NW
VOCAB = 1000000


def _positional_encoding_np(position, d_model):
    pos = np.arange(position)[:, np.newaxis].astype(np.float64)
    i = np.arange(d_model)[np.newaxis, :].astype(np.float64)
    angle_rates = 1.0 / np.power(10000.0, 2.0 * (i // 2) / np.float32(d_model))
    angle_rads = pos * angle_rates
    angle_rads[:, 0::2] = np.sin(angle_rads[:, 0::2])
    angle_rads[:, 1::2] = np.cos(angle_rads[:, 1::2])
    return angle_rads.astype(np.float32)


def _sc_body(xt_hbm, tp_hbm, pos_hbm, out_hbm, idx_slab, pos_v,
             gbuf0, gbuf1, wbuf0, wbuf1, gsem0, gsem1, wsem0, wsem1):
    wid = lax.axis_index("s") * NC + lax.axis_index("c")
    b0 = wid * BCH
    gbufs = (gbuf0, gbuf1)
    wbufs = (wbuf0, wbuf1)
    gsems = (gsem0, gsem1)
    wsems = (wsem0, wsem1)

    pltpu.sync_copy(xt_hbm.at[:, pl.ds(b0, BCH)], idx_slab)
    pltpu.sync_copy(pos_hbm, pos_v)

    def start_gather(s, slot):
        pltpu.async_copy(tp_hbm.at[idx_slab.at[s]], gbufs[slot], gsems[slot])

    def wait_gather(s, slot):
        pltpu.make_async_copy(tp_hbm.at[idx_slab.at[s]], gbufs[slot],
                              gsems[slot]).wait()

    def start_write(s, slot):
        pltpu.async_copy(wbufs[slot], out_hbm.at[s, pl.ds(b0, BCH)],
                         wsems[slot])

    def wait_write(s, slot):
        pltpu.make_async_copy(wbufs[slot], out_hbm.at[s, pl.ds(b0, BCH)],
                              wsems[slot]).wait()

    def compute(s, slot):
        g, w = gbufs[slot], wbufs[slot]
        pq = [pos_v[s, pl.ds(16 * q, 16)] for q in range(4)]

        def b_body(b, carry):
            for q in range(4):
                sl = pl.ds(16 * q, 16)
                w[b, sl] = g[b, sl] * SCALE + pq[q]
            return carry

        lax.fori_loop(0, BCH, b_body, 0, unroll=8)

    start_gather(0, 0)
    start_gather(1, 1)

    for slot in range(2):
        wait_gather(slot, slot)
        compute(slot, slot)
        start_write(slot, slot)
        start_gather(slot + 2, slot)

    def group(k, carry):
        for slot in range(2):
            s = 2 * k + slot
            wait_gather(s, slot)
            wait_write(s - 2, slot)
            compute(s, slot)
            start_write(s, slot)
            start_gather(s + 2, slot)
        return carry

    lax.fori_loop(1, MAXLEN // 2 - 1, group, 0)

    for slot in range(2):
        s = MAXLEN - 2 + slot
        wait_gather(s, slot)
        wait_write(s - 2, slot)
        compute(s, slot)
        start_write(s, slot)

    wait_write(MAXLEN - 2, 0)
    wait_write(MAXLEN - 1, 1)


@jax.jit
def kernel(x, token_table):
    posc = jnp.asarray(_positional_encoding_np(MAXLEN, EMBED_DIM))

    xt = jnp.transpose(x.astype(jnp.int32))
    tpad = jnp.pad(token_table, ((0, 0), (0, 128 - EMBED_DIM)))

    mesh = plsc.VectorSubcoreMesh(core_axis_name="c", subcore_axis_name="s")
    fn = pl.kernel(
        _sc_body,
        out_type=jax.ShapeDtypeStruct((MAXLEN, B, EMBED_DIM), jnp.float32),
        mesh=mesh,
        scratch_types=[
            pltpu.VMEM((MAXLEN, BCH), jnp.int32),
            pltpu.VMEM((MAXLEN, EMBED_DIM), jnp.float32),
            pltpu.VMEM((BCH, 128), jnp.float32),
            pltpu.VMEM((BCH, 128), jnp.float32),
            pltpu.VMEM((BCH, EMBED_DIM), jnp.float32),
            pltpu.VMEM((BCH, EMBED_DIM), jnp.float32),
            pltpu.SemaphoreType.DMA,
            pltpu.SemaphoreType.DMA,
            pltpu.SemaphoreType.DMA,
            pltpu.SemaphoreType.DMA,
        ],
        compiler_params=pltpu.CompilerParams(use_tc_tiling_on_sc=False),
    )
    out3 = fn(xt, tpad, posc)
    return jnp.transpose(out3, (1, 0, 2))
